# baseline (speedup 1.0000x reference)
import functools

import jax
import jax.numpy as jnp
from jax import lax
from jax.experimental import pallas as pl
from jax.experimental.pallas import tpu as pltpu
from jax.experimental.pallas import tpu_sc as plsc

N_NODES = 10000
D = 128
N_EDGES = 320000

NUM_CORES = 2
NUM_SUBCORES = 16
NW = NUM_CORES * NUM_SUBCORES

G = 80
NCHUNK = N_EDGES // (NW * G)
PADN = 10240
STRIPE = PADN // NUM_SUBCORES

NBUF = 2
NPHASE = 5
HC = NCHUNK // NPHASE


def _sc_agg_body(x_hbm, edge_hbm, out_hbm, src_v, dst_v, rows_v, acc, sems,
                 ssems, isems):
  c = lax.axis_index("c")
  s = lax.axis_index("s")
  wid = s * NUM_CORES + c

  zz = jnp.zeros((16,), jnp.float32)

  def zero_row(i, _):
    for j in range(D // 16):
      rows_v[0, i, pl.ds(j * 16, 16)] = zz
    return 0

  with jax.named_scope("zero_acc"):
    lax.fori_loop(0, G, zero_row, 0)
    for k in range(STRIPE // G):
      pltpu.sync_copy(rows_v.at[0], acc.at[pl.ds(s * STRIPE + k * G, G)])
    plsc.subcore_barrier()

  def gather(j, b, i):
    return pltpu.async_copy(x_hbm.at[src_v.at[i, j]], rows_v.at[b],
                            sems.at[b])

  def wait_gather(j, b, i):
    pltpu.make_async_copy(x_hbm.at[src_v.at[i, j]], rows_v.at[b],
                          sems.at[b]).wait()

  def scatter(j, b, i):
    pltpu.async_copy(rows_v.at[b], acc.at[dst_v.at[i, j]], ssems.at[b],
                     add=True)

  def wait_scatter(b, i):
    pltpu.make_async_copy(rows_v.at[b], acc.at[dst_v.at[i, 0]],
                          ssems.at[b]).wait()

  def stage_idx(p, i):
    pltpu.async_copy(edge_hbm.at[0, wid, p], src_v.at[i], isems.at[i])
    pltpu.async_copy(edge_hbm.at[1, wid, p], dst_v.at[i], isems.at[i])

  def wait_idx(i):
    pltpu.make_async_copy(edge_hbm.at[0, wid, 0], src_v.at[i],
                          isems.at[i]).wait()
    pltpu.make_async_copy(edge_hbm.at[1, wid, 0], dst_v.at[i],
                          isems.at[i]).wait()

  stage_idx(0, 0)
  for p in range(NPHASE):
    i = p % 2
    with jax.named_scope("idx_stage"):
      wait_idx(i)
      if p + 1 < NPHASE:
        stage_idx(p + 1, 1 - i)

    with jax.named_scope("edge_loop"):
      for b in range(NBUF):
        gather(b, b, i)

      nfull = (HC - NBUF) // NBUF

      def step(jo, _):
        for b in range(NBUF):
          jj = jo * NBUF + b
          wait_gather(jj, b, i)
          scatter(jj, b, i)
        for b in range(NBUF):
          jj = jo * NBUF + b
          wait_scatter(b, i)
          gather(jj + NBUF, b, i)
        return 0

      lax.fori_loop(0, nfull, step, 0)
      for jj in range(nfull * NBUF, HC):
        b = jj % NBUF
        wait_gather(jj, b, i)
        scatter(jj, b, i)
        if jj + NBUF < HC:
          wait_scatter(b, i)
          gather(jj + NBUF, b, i)
      for jj in range(max(nfull * NBUF, HC - NBUF), HC):
        wait_scatter(jj % NBUF, i)
  plsc.subcore_barrier()

  with jax.named_scope("flush"):
    base = s * STRIPE
    pltpu.sync_copy(acc.at[pl.ds(base, STRIPE)],
                    out_hbm.at[c, pl.ds(base, STRIPE)])


_sc_agg = functools.partial(
    pl.kernel,
    out_type=jax.ShapeDtypeStruct((NUM_CORES, PADN, D), jnp.float32),
    mesh=plsc.VectorSubcoreMesh(core_axis_name="c", subcore_axis_name="s"),
    scratch_types=[
        pltpu.VMEM((2, HC, G), jnp.int32),
        pltpu.VMEM((2, HC, G), jnp.int32),
        pltpu.VMEM((NBUF, G, D), jnp.float32),
        pltpu.VMEM_SHARED((PADN, D), jnp.float32),
        pltpu.SemaphoreType.DMA((NBUF,)),
        pltpu.SemaphoreType.DMA((NBUF,)),
        pltpu.SemaphoreType.DMA((2,)),
    ],
)(_sc_agg_body)


def _tc_epilogue_body(p_ref, w_ref, o_ref):
  h = p_ref[0] + p_ref[1]
  h = jnp.dot(h, w_ref[...], preferred_element_type=jnp.float32)
  o_ref[...] = jnp.where(h > 0, h, jnp.exp(jnp.minimum(h, 0.0)) - 1.0)


_BR = 1000


@jax.jit
def kernel(x, edge_index, W):
  edges = edge_index.astype(jnp.int32).reshape(2, NW, NPHASE, HC, G)

  partials = _sc_agg(x, edges)

  out = pl.pallas_call(
      _tc_epilogue_body,
      grid=(N_NODES // _BR,),
      in_specs=[
          pl.BlockSpec((NUM_CORES, _BR, D), lambda i: (0, i, 0)),
          pl.BlockSpec((D, D), lambda i: (0, 0)),
      ],
      out_specs=pl.BlockSpec((_BR, D), lambda i: (i, 0)),
      out_shape=jax.ShapeDtypeStruct((N_NODES, D), jnp.float32),
  )(partials, W)
  return out


# speedup vs baseline: 1.3778x; 1.3778x over previous
import functools

import jax
import jax.numpy as jnp
from jax import lax
from jax.experimental import pallas as pl
from jax.experimental.pallas import tpu as pltpu
from jax.experimental.pallas import tpu_sc as plsc

N_NODES = 10000
D = 128
N_EDGES = 320000

NUM_CORES = 2
NUM_SUBCORES = 16
NW = NUM_CORES * NUM_SUBCORES

G = 80
NCHUNK = N_EDGES // (NW * G)
PADN = 10240
STRIPE = PADN // NUM_SUBCORES

NBUF = 3
NPHASE = 5
HC = NCHUNK // NPHASE


def _sc_agg_body(x_hbm, edge_hbm, out_hbm, src_v, dst_v, rows_v, acc, sems,
                 isems):
  c = lax.axis_index("c")
  s = lax.axis_index("s")
  wid = s * NUM_CORES + c

  zz = jnp.zeros((16,), jnp.float32)

  def zero_row(i, _):
    for j in range(D // 16):
      rows_v[0, i, pl.ds(j * 16, 16)] = zz
    return 0

  with jax.named_scope("zero_acc"):
    lax.fori_loop(0, G, zero_row, 0)
    for k in range(STRIPE // G):
      pltpu.sync_copy(rows_v.at[0], acc.at[pl.ds(s * STRIPE + k * G, G)])
    plsc.subcore_barrier()

  def gather(j, b, i):
    return pltpu.async_copy(x_hbm.at[src_v.at[i, j]], rows_v.at[b],
                            sems.at[b])

  def wait_gather(j, b, i):
    pltpu.make_async_copy(x_hbm.at[src_v.at[i, j]], rows_v.at[b],
                          sems.at[b]).wait()

  def scatter(j, b, i):
    pltpu.sync_copy(rows_v.at[b], acc.at[dst_v.at[i, j]], add=True)

  def stage_idx(p, i):
    pltpu.async_copy(edge_hbm.at[0, wid, p], src_v.at[i], isems.at[i])
    pltpu.async_copy(edge_hbm.at[1, wid, p], dst_v.at[i], isems.at[i])

  def wait_idx(i):
    pltpu.make_async_copy(edge_hbm.at[0, wid, 0], src_v.at[i],
                          isems.at[i]).wait()
    pltpu.make_async_copy(edge_hbm.at[1, wid, 0], dst_v.at[i],
                          isems.at[i]).wait()

  stage_idx(0, 0)
  for p in range(NPHASE):
    i = p % 2
    with jax.named_scope("idx_stage"):
      wait_idx(i)
      if p + 1 < NPHASE:
        stage_idx(p + 1, 1 - i)

    with jax.named_scope("edge_loop"):
      for b in range(NBUF):
        gather(b, b, i)

      nfull = (HC - NBUF) // NBUF

      def step(jo, _):
        for b in range(NBUF):
          jj = jo * NBUF + b
          wait_gather(jj, b, i)
          scatter(jj, b, i)
          gather(jj + NBUF, b, i)
        return 0

      lax.fori_loop(0, nfull, step, 0)
      for jj in range(nfull * NBUF, HC):
        b = jj % NBUF
        wait_gather(jj, b, i)
        scatter(jj, b, i)
        if jj + NBUF < HC:
          gather(jj + NBUF, b, i)
  plsc.subcore_barrier()

  with jax.named_scope("flush"):
    base = s * STRIPE
    pltpu.sync_copy(acc.at[pl.ds(base, STRIPE)],
                    out_hbm.at[c, pl.ds(base, STRIPE)])


_sc_agg = functools.partial(
    pl.kernel,
    out_type=jax.ShapeDtypeStruct((NUM_CORES, PADN, D), jnp.float32),
    mesh=plsc.VectorSubcoreMesh(core_axis_name="c", subcore_axis_name="s"),
    scratch_types=[
        pltpu.VMEM((2, HC, G), jnp.int32),
        pltpu.VMEM((2, HC, G), jnp.int32),
        pltpu.VMEM((NBUF, G, D), jnp.float32),
        pltpu.VMEM_SHARED((PADN, D), jnp.float32),
        pltpu.SemaphoreType.DMA((NBUF,)),
        pltpu.SemaphoreType.DMA((2,)),
    ],
)(_sc_agg_body)


def _tc_epilogue_body(p_ref, w_ref, o_ref):
  h = p_ref[0] + p_ref[1]
  h = jnp.dot(h, w_ref[...], preferred_element_type=jnp.float32)
  o_ref[...] = jnp.where(h > 0, h, jnp.exp(jnp.minimum(h, 0.0)) - 1.0)


_BR = 1000


@jax.jit
def kernel(x, edge_index, W):
  edges = edge_index.astype(jnp.int32).reshape(2, NW, NPHASE, HC, G)

  partials = _sc_agg(x, edges)

  out = pl.pallas_call(
      _tc_epilogue_body,
      grid=(N_NODES // _BR,),
      in_specs=[
          pl.BlockSpec((NUM_CORES, _BR, D), lambda i: (0, i, 0)),
          pl.BlockSpec((D, D), lambda i: (0, 0)),
      ],
      out_specs=pl.BlockSpec((_BR, D), lambda i: (i, 0)),
      out_shape=jax.ShapeDtypeStruct((N_NODES, D), jnp.float32),
  )(partials, W)
  return out
